# Initial kernel scaffold; baseline (speedup 1.0000x reference)
import jax
import jax.numpy as jnp
from jax.experimental import pallas as pl


def kernel(x, A_tilde_index, A_tilde_weight, s1_sct_index, s1_sct_weight, s2_sct_index, s2_sct_weight, s3_sct_index, s3_sct_weight, adj_p_index, adj_p_weight, W, a, Wg, bg):
    raise NotImplementedError("write your pallas kernel here")


# speedup vs baseline: 17.7106x; 17.7106x over previous
import functools

import jax
import jax.numpy as jnp
from jax import lax
from jax.experimental import pallas as pl
from jax.experimental.pallas import tpu as pltpu
from jax.experimental.pallas import tpu_sc as plsc

N = 10000
E = 320000
NFEAT = 128
HID = 16
NHEADS = 8
NCLASS = 10
SMOO = 0.5

C = 128
NCHUNK = E // C
NTILE = 16
NSC = 2
RPT = 624
TAILN = N - RPT * NTILE
DPAD = 16

_MESH = plsc.VectorSubcoreMesh(core_axis_name="c", subcore_axis_name="s")


@functools.partial(
    pl.kernel,
    out_type=jax.ShapeDtypeStruct((4, N, NFEAT), jnp.float32),
    mesh=_MESH,
    scratch_types=[
        pltpu.VMEM_SHARED((N, NFEAT), jnp.float32),
        pltpu.VMEM((C, NFEAT), jnp.float32),
        pltpu.VMEM((C,), jnp.int32),
        pltpu.VMEM((C,), jnp.int32),
        pltpu.VMEM((C,), jnp.float32),
        pltpu.SemaphoreType.DMA,
    ],
)
def _spmm4_sc(h_hbm, src_hbm, dst_hbm, w_hbm, zeros_hbm, out_hbm,
              acc, rows, srcb, dstb, wb, sem):
    cid = lax.axis_index("c")
    sid = lax.axis_index("s")
    row0 = sid * RPT
    ntrips = jnp.where(sid < NCHUNK - (NCHUNK // NTILE) * NTILE,
                       NCHUNK // NTILE + 1, NCHUNK // NTILE)

    for m_local in range(2):
        m = cid * 2 + m_local

        pltpu.sync_copy(zeros_hbm.at[pl.ds(row0, RPT), :],
                        acc.at[pl.ds(row0, RPT), :])

        @pl.when(sid == NTILE - 1)
        def _zero_tail():
            pltpu.sync_copy(zeros_hbm.at[pl.ds(RPT * NTILE, TAILN), :],
                            acc.at[pl.ds(RPT * NTILE, TAILN), :])

        plsc.subcore_barrier()

        def chunk_body(k, _):
            base = (k * NTILE + sid) * C
            pltpu.sync_copy(src_hbm.at[m, pl.ds(base, C)], srcb)
            pltpu.sync_copy(dst_hbm.at[m, pl.ds(base, C)], dstb)
            pltpu.sync_copy(w_hbm.at[m, pl.ds(base, C)], wb)
            pltpu.async_copy(h_hbm.at[srcb], rows, sem).wait()

            def group_body(g, carry):
                e0 = g * 16
                wv = wb[pl.ds(e0, 16)]
                for j in range(16):
                    w = wv[j]
                    for f in range(NFEAT // 16):
                        sl = pl.ds(f * 16, 16)
                        rows[e0 + j, sl] = rows[e0 + j, sl] * w
                return carry

            lax.fori_loop(0, C // 16, group_body, 0)
            pltpu.sync_copy(rows, acc.at[dstb], add=True)
            return _

        lax.fori_loop(0, ntrips, chunk_body, 0)
        plsc.subcore_barrier()

        pltpu.sync_copy(acc.at[pl.ds(row0, RPT), :],
                        out_hbm.at[m, pl.ds(row0, RPT), :])

        @pl.when(sid == NTILE - 1)
        def _out_tail():
            pltpu.sync_copy(acc.at[pl.ds(RPT * NTILE, TAILN), :],
                            out_hbm.at[m, pl.ds(RPT * NTILE, TAILN), :])

        plsc.subcore_barrier()


@functools.partial(
    pl.kernel,
    out_type=jax.ShapeDtypeStruct((NSC, N, NFEAT), jnp.float32),
    mesh=_MESH,
    scratch_types=[
        pltpu.VMEM_SHARED((N, NFEAT), jnp.float32),
        pltpu.VMEM((C, NFEAT), jnp.float32),
        pltpu.VMEM((C,), jnp.int32),
        pltpu.VMEM((C,), jnp.int32),
        pltpu.VMEM((C,), jnp.float32),
        pltpu.SemaphoreType.DMA,
    ],
)
def _spmm_adj_sc(sup_hbm, src_hbm, dst_hbm, w_hbm, zeros_hbm, out_hbm,
                 acc, rows, srcb, dstb, wb, sem):
    cid = lax.axis_index("c")
    sid = lax.axis_index("s")
    wid = sid * NSC + cid
    row0 = sid * RPT
    nw = NSC * NTILE
    ntrips = jnp.where(wid < NCHUNK - (NCHUNK // nw) * nw,
                       NCHUNK // nw + 1, NCHUNK // nw)

    pltpu.sync_copy(zeros_hbm.at[pl.ds(row0, RPT), :],
                    acc.at[pl.ds(row0, RPT), :])

    @pl.when(sid == NTILE - 1)
    def _zero_tail():
        pltpu.sync_copy(zeros_hbm.at[pl.ds(RPT * NTILE, TAILN), :],
                        acc.at[pl.ds(RPT * NTILE, TAILN), :])

    plsc.subcore_barrier()

    def chunk_body(k, _):
        base = (k * nw + wid) * C
        pltpu.sync_copy(src_hbm.at[pl.ds(base, C)], srcb)
        pltpu.sync_copy(dst_hbm.at[pl.ds(base, C)], dstb)
        pltpu.sync_copy(w_hbm.at[pl.ds(base, C)], wb)
        pltpu.async_copy(sup_hbm.at[srcb], rows, sem).wait()

        def group_body(g, carry):
            e0 = g * 16
            wv = wb[pl.ds(e0, 16)]
            for j in range(16):
                w = wv[j]
                for f in range(NFEAT // 16):
                    sl = pl.ds(f * 16, 16)
                    rows[e0 + j, sl] = rows[e0 + j, sl] * w
            return carry

        lax.fori_loop(0, C // 16, group_body, 0)
        pltpu.sync_copy(rows, acc.at[dstb], add=True)
        return _

    lax.fori_loop(0, ntrips, chunk_body, 0)
    plsc.subcore_barrier()

    pltpu.sync_copy(acc.at[pl.ds(row0, RPT), :],
                    out_hbm.at[cid, pl.ds(row0, RPT), :])

    @pl.when(sid == NTILE - 1)
    def _out_tail():
        pltpu.sync_copy(acc.at[pl.ds(RPT * NTILE, TAILN), :],
                        out_hbm.at[cid, pl.ds(RPT * NTILE, TAILN), :])


_BN = 2000


def _mm_body(x_ref, w_ref, o_ref):
    o_ref[:, :] = jnp.dot(x_ref[:, :], w_ref[:, :],
                          preferred_element_type=jnp.float32)


def _attn_body(c0r, c1r, c2r, c3r, a_ref, r_ref, xcat_ref):
    chans = (c0r[0], jnp.abs(c1r[0]), jnp.abs(c2r[0]), jnp.abs(c3r[0]))
    es = []
    for c in range(4):
        e = jnp.dot(chans[c], a_ref[c], preferred_element_type=jnp.float32)
        es.append(jnp.where(e > 0, e, 0.2 * e))
    mx = jnp.maximum(jnp.maximum(es[0], es[1]), jnp.maximum(es[2], es[3]))
    ex = [jnp.exp(e - mx) for e in es]
    denom = ex[0] + ex[1] + ex[2] + ex[3]
    out = jnp.zeros_like(chans[0])
    for c in range(4):
        attn = ex[c] / denom
        out = out + jnp.dot(attn, r_ref[:, :],
                            preferred_element_type=jnp.float32) * chans[c]
    xcat_ref[:, :] = jnp.maximum(out, 0.0)


def _final_body(p0r, p1r, xcat_ref, wg_ref, bg_ref, o_ref):
    z = p0r[0] + p1r[0] + SMOO * xcat_ref[:, :]
    logits = (jnp.dot(z, wg_ref[:, :], preferred_element_type=jnp.float32)
              / (1.0 + SMOO)) + bg_ref[0, :][None, :]
    mx = jnp.max(logits, axis=1, keepdims=True)
    lse = mx + jnp.log(jnp.sum(jnp.exp(logits - mx), axis=1, keepdims=True))
    o_ref[:, :] = logits - lse


def kernel(x, A_tilde_index, A_tilde_weight, s1_sct_index, s1_sct_weight,
           s2_sct_index, s2_sct_weight, s3_sct_index, s3_sct_weight,
           adj_p_index, adj_p_weight, W, a, Wg, bg):
    f32 = jnp.float32

    W_all = W.transpose(1, 0, 2).reshape(NFEAT, NHEADS * HID)
    src4 = jnp.stack([A_tilde_index[0], s1_sct_index[0],
                      s2_sct_index[0], s3_sct_index[0]])
    dst4 = jnp.stack([A_tilde_index[1], s1_sct_index[1],
                      s2_sct_index[1], s3_sct_index[1]])
    w4 = jnp.stack([A_tilde_weight, s1_sct_weight,
                    s2_sct_weight, s3_sct_weight])
    amat = (a.transpose(1, 0, 2)[:, :, :, None]
            * jnp.eye(NHEADS, dtype=f32)[None, :, None, :])
    amat = amat.reshape(4, NHEADS * HID, NHEADS)
    rmat = jnp.repeat(jnp.eye(NHEADS, dtype=f32), HID, axis=1)
    wg_pad = jnp.pad(Wg, ((0, 0), (0, DPAD - NCLASS)))
    bg_row = jnp.concatenate(
        [bg, jnp.full((DPAD - NCLASS,), -jnp.inf, dtype=f32)]).reshape(1, DPAD)
    z128 = jnp.zeros((N, NFEAT), dtype=f32)

    h = pl.pallas_call(
        _mm_body,
        grid=(N // _BN,),
        in_specs=[pl.BlockSpec((_BN, NFEAT), lambda i: (i, 0)),
                  pl.BlockSpec((NFEAT, NFEAT), lambda i: (0, 0))],
        out_specs=pl.BlockSpec((_BN, NFEAT), lambda i: (i, 0)),
        out_shape=jax.ShapeDtypeStruct((N, NFEAT), f32),
    )(x, W_all)

    c4 = _spmm4_sc(h, src4, dst4, w4, z128)

    def _csel(m):
        return pl.BlockSpec((1, _BN, NFEAT), lambda i, m=m: (m, i, 0))

    xcat = pl.pallas_call(
        _attn_body,
        grid=(N // _BN,),
        in_specs=[_csel(0), _csel(1), _csel(2), _csel(3),
                  pl.BlockSpec((4, NHEADS * HID, NHEADS), lambda i: (0, 0, 0)),
                  pl.BlockSpec((NHEADS, NHEADS * HID), lambda i: (0, 0))],
        out_specs=pl.BlockSpec((_BN, NFEAT), lambda i: (i, 0)),
        out_shape=jax.ShapeDtypeStruct((N, NFEAT), f32),
    )(c4, c4, c4, c4, amat, rmat)

    p2 = _spmm_adj_sc(xcat, adj_p_index[0], adj_p_index[1],
                      adj_p_weight, z128)

    out16 = pl.pallas_call(
        _final_body,
        grid=(N // _BN,),
        in_specs=[pl.BlockSpec((1, _BN, NFEAT), lambda i: (0, i, 0)),
                  pl.BlockSpec((1, _BN, NFEAT), lambda i: (1, i, 0)),
                  pl.BlockSpec((_BN, NFEAT), lambda i: (i, 0)),
                  pl.BlockSpec((NHEADS * HID, DPAD), lambda i: (0, 0)),
                  pl.BlockSpec((1, DPAD), lambda i: (0, 0))],
        out_specs=pl.BlockSpec((_BN, DPAD), lambda i: (i, 0)),
        out_shape=jax.ShapeDtypeStruct((N, DPAD), f32),
    )(p2, p2, xcat, wg_pad, bg_row)

    return out16[:, :NCLASS]
